# baseline (speedup 1.0000x reference)
import functools
import jax
import jax.numpy as jnp
from jax import lax
from jax.experimental import pallas as pl
from jax.experimental.pallas import tpu as pltpu
from jax.experimental.pallas import tpu_sc as plsc

_NC, _NS = 2, 16
_NW = _NC * _NS

_N, _C, _H, _W = 32, 192, 56, 56
_COLS = (_C // 2) * _H * _W
_TPB = 2 * _COLS
_NCHUNK = 8
_CH = _COLS // _NCHUNK
_NBUF = 3


def _sc_body(z_hbm, o1_hbm, o2_hbm, *scratch):
    bufs = scratch[:_NBUF]
    isems = scratch[_NBUF:2 * _NBUF]
    osems = scratch[2 * _NBUF:3 * _NBUF]
    b = lax.axis_index("s") * _NC + lax.axis_index("c")

    units = [(h, c) for h in (0, 1) for c in range(_NCHUNK)]
    nu = len(units)

    def in_copy(i):
        h, c = units[i]
        src = z_hbm.at[pl.ds(b * _TPB + h * _COLS + c * _CH, _CH)]
        return pltpu.make_async_copy(src, bufs[i % _NBUF], isems[i % _NBUF])

    def out_copy(i):
        h, c = units[i]
        dst_ref = o1_hbm if h == 0 else o2_hbm
        dst = dst_ref.at[pl.ds(b * _COLS + c * _CH, _CH)]
        return pltpu.make_async_copy(bufs[i % _NBUF], dst, osems[i % _NBUF])

    for i in range(min(_NBUF, nu)):
        in_copy(i).start()
    for i in range(nu):
        in_copy(i).wait()
        out_copy(i).start()
        if i + _NBUF < nu:
            out_copy(i).wait()
            in_copy(i + _NBUF).start()
    for i in range(nu - _NBUF, nu):
        if i >= 0:
            out_copy(i).wait()


def kernel(z):
    n, c, h, w = z.shape
    ch = c // 2
    zf = z.reshape(n * 2 * _COLS)
    mesh = plsc.VectorSubcoreMesh(core_axis_name="c", subcore_axis_name="s")
    out1, out2 = pl.kernel(
        _sc_body,
        out_type=[
            jax.ShapeDtypeStruct((n * _COLS,), z.dtype),
            jax.ShapeDtypeStruct((n * _COLS,), z.dtype),
        ],
        mesh=mesh,
        scratch_types=(
            [pltpu.VMEM((_CH,), jnp.float32)] * _NBUF
            + [pltpu.SemaphoreType.DMA] * (2 * _NBUF)
        ),
    )(zf)
    z1 = out1.reshape(n, ch, h, w)
    z2 = out2.reshape(n, ch, h, w)
    log_det = jnp.zeros((), z.dtype)
    return (z1, z2, log_det)
